# baseline (speedup 1.0000x reference)
import functools

import jax
import jax.numpy as jnp
from jax import lax
from jax.experimental import pallas as pl
from jax.experimental.pallas import tpu as pltpu
from jax.experimental.pallas import tpu_sc as plsc

P, Q, S = 64, 64, 4096
L = 16
NC, NS = 2, 16
NW = NC * NS
P_TC = 32
P_SC = P - P_TC
QB = 8
NCHUNK = Q // QB
SCH = S // L

_mesh = plsc.VectorSubcoreMesh(core_axis_name="c", subcore_axis_name="s")


@functools.partial(
    pl.kernel,
    mesh=_mesh,
    out_type=jax.ShapeDtypeStruct((P_SC, Q), jnp.float32),
    compiler_params=pltpu.CompilerParams(needs_layout_passes=False),
    scratch_types=[
        pltpu.VMEM((S,), jnp.float32),
        pltpu.VMEM((P,), jnp.float32),
        pltpu.VMEM((Q,), jnp.float32),
        pltpu.VMEM((QB, S), jnp.float32),
        pltpu.VMEM((QB, S), jnp.float32),
        pltpu.SemaphoreType.DMA,
        pltpu.SemaphoreType.DMA,
    ],
)
def _sc_logits(costs_hbm, valid_hbm, occ_hbm, out_hbm,
               costs_v, valid_v, out_v, buf0, buf1, sem0, sem1):
    wid = lax.axis_index("s") * NC + lax.axis_index("c")
    pltpu.sync_copy(valid_hbm, valid_v)
    bufs = (buf0, buf1)
    sems = (sem0, sem1)
    lane_iota = lax.iota(jnp.int32, L)
    for t in range((P_SC + NW - 1) // NW):

        @pl.when(t * NW + wid < P_SC)
        def _problem():
            pp = t * NW + wid
            p = P_TC + pp
            pltpu.sync_copy(costs_hbm.at[p], costs_v)

            def _clean(si, carry):
                base = pl.multiple_of(si * L, L)
                c = costs_v[pl.ds(base, L)]
                costs_v[pl.ds(base, L)] = jnp.where(c != c, jnp.float32(1.0), c)
                return carry

            lax.fori_loop(0, SCH, _clean, 0)

            pbase = pl.multiple_of((p // L) * L, L)
            vvec = valid_v[pl.ds(pbase, L)]
            vfv = jnp.sum(jnp.where(lane_iota == (p - pbase), vvec,
                                    jnp.float32(0.0)))

            def _compute8(buf, res, off):
                def _acc(si, accs):
                    base = pl.multiple_of(si * L, L)
                    c = costs_v[pl.ds(base, L)]
                    return tuple(accs[j] + buf[j, pl.ds(base, L)] * c
                                 for j in range(QB))

                accs = lax.fori_loop(
                    0, SCH, _acc,
                    tuple(jnp.zeros((L,), jnp.float32) for _ in range(QB)))
                for j in range(QB):
                    res = jnp.where(lane_iota == (off + j),
                                    jnp.sum(accs[j]), res)
                return res

            pltpu.async_copy(occ_hbm.at[p, pl.ds(0, QB), :], buf0, sem0)

            def _pair(g, carry):
                q0 = pl.multiple_of(g * 2 * QB, 2 * QB)
                h1 = pltpu.async_copy(
                    occ_hbm.at[p, pl.ds(q0 + QB, QB), :], buf1, sem1)
                pltpu.make_async_copy(
                    occ_hbm.at[p, pl.ds(q0, QB), :], buf0, sem0).wait()
                res = _compute8(buf0, jnp.zeros((L,), jnp.float32), 0)

                @pl.when(g < NCHUNK // 2 - 1)
                def _prefetch():
                    pltpu.async_copy(
                        occ_hbm.at[p, pl.ds(q0 + 2 * QB, QB), :], buf0, sem0)

                h1.wait()
                res = _compute8(buf1, res, QB)
                out_v[pl.ds(pl.multiple_of(g * L, L), L)] = res * vfv
                return carry

            lax.fori_loop(0, NCHUNK // 2, _pair, 0)
            pltpu.sync_copy(out_v, out_hbm.at[pp])


PB_TC = 4


def _tc_body(valid_ref, costs_ref, occ_ref, out_ref):
    pid = pl.program_id(0)
    for i in range(PB_TC):
        c = costs_ref[pid * PB_TC + i]
        c = jnp.where(jnp.isnan(c), jnp.float32(1.0), c)
        vf = valid_ref[pid * PB_TC + i]
        out_ref[i] = jnp.sum(occ_ref[i] * c[None, :], axis=1)[None, :] * vf


def _tc_logits(costs, valid_f, occ):
    out = pl.pallas_call(
        _tc_body,
        grid=(P_TC // PB_TC,),
        in_specs=[
            pl.BlockSpec(memory_space=pltpu.SMEM),
            pl.BlockSpec((P, S), lambda p: (0, 0)),
            pl.BlockSpec((PB_TC, Q, S), lambda p: (p, 0, 0)),
        ],
        out_specs=pl.BlockSpec((PB_TC, 1, Q), lambda p: (p, 0, 0)),
        out_shape=jax.ShapeDtypeStruct((P_TC, 1, Q), jnp.float32),
    )(valid_f, costs, occ)
    return out.reshape(P_TC, Q)


def kernel(costs, valid, occ):
    valid_f = valid.astype(jnp.float32)
    sc_part = _sc_logits(costs, valid_f, occ)
    tc_part = _tc_logits(costs, valid_f, occ)
    logits = jnp.concatenate([tc_part, sc_part], axis=0)
    return (logits, valid)


# speedup vs baseline: 1.0107x; 1.0107x over previous
import functools

import jax
import jax.numpy as jnp
from jax import lax
from jax.experimental import pallas as pl
from jax.experimental.pallas import tpu as pltpu
from jax.experimental.pallas import tpu_sc as plsc

P, Q, S = 64, 64, 4096
L = 16
NC, NS = 2, 16
NW = NC * NS
P_TC = 32
P_SC = P - P_TC
QB = 8
NCHUNK = Q // QB
SCH = S // L

_mesh = plsc.VectorSubcoreMesh(core_axis_name="c", subcore_axis_name="s")


@functools.partial(
    pl.kernel,
    mesh=_mesh,
    out_type=jax.ShapeDtypeStruct((P_SC, Q), jnp.float32),
    compiler_params=pltpu.CompilerParams(needs_layout_passes=False),
    scratch_types=[
        pltpu.VMEM((S,), jnp.float32),
        pltpu.VMEM((P,), jnp.float32),
        pltpu.VMEM((Q,), jnp.float32),
        pltpu.VMEM((QB, S), jnp.float32),
        pltpu.VMEM((QB, S), jnp.float32),
        pltpu.SemaphoreType.DMA,
        pltpu.SemaphoreType.DMA,
    ],
)
def _sc_logits(costs_hbm, valid_hbm, occ_hbm, out_hbm,
               costs_v, valid_v, out_v, buf0, buf1, sem0, sem1):
    wid = lax.axis_index("s") * NC + lax.axis_index("c")
    pltpu.sync_copy(valid_hbm, valid_v)
    bufs = (buf0, buf1)
    sems = (sem0, sem1)
    lane_iota = lax.iota(jnp.int32, L)
    for t in range((P_SC + NW - 1) // NW):

        @pl.when(t * NW + wid < P_SC)
        def _problem():
            pp = t * NW + wid
            p = P_TC + pp
            pltpu.sync_copy(costs_hbm.at[p], costs_v)

            def _clean(si, carry):
                base = pl.multiple_of(si * L, L)
                c = costs_v[pl.ds(base, L)]
                costs_v[pl.ds(base, L)] = jnp.where(c != c, jnp.float32(1.0), c)
                return carry

            lax.fori_loop(0, SCH, _clean, 0)

            pbase = pl.multiple_of((p // L) * L, L)
            vvec = valid_v[pl.ds(pbase, L)]
            vfv = jnp.sum(jnp.where(lane_iota == (p - pbase), vvec,
                                    jnp.float32(0.0)))

            def _compute8(buf, res, off):
                def _acc(si, accs):
                    base = pl.multiple_of(si * L, L)
                    c = costs_v[pl.ds(base, L)]
                    return tuple(accs[j] + buf[j, pl.ds(base, L)] * c
                                 for j in range(QB))

                accs = lax.fori_loop(
                    0, SCH, _acc,
                    tuple(jnp.zeros((L,), jnp.float32) for _ in range(QB)))
                for j in range(QB):
                    res = jnp.where(lane_iota == (off + j),
                                    jnp.sum(accs[j]), res)
                return res

            pltpu.async_copy(occ_hbm.at[p, pl.ds(0, QB), :], buf0, sem0)

            def _pair(g, carry):
                q0 = pl.multiple_of(g * 2 * QB, 2 * QB)
                h1 = pltpu.async_copy(
                    occ_hbm.at[p, pl.ds(q0 + QB, QB), :], buf1, sem1)
                pltpu.make_async_copy(
                    occ_hbm.at[p, pl.ds(q0, QB), :], buf0, sem0).wait()
                res = _compute8(buf0, jnp.zeros((L,), jnp.float32), 0)

                @pl.when(g < NCHUNK // 2 - 1)
                def _prefetch():
                    pltpu.async_copy(
                        occ_hbm.at[p, pl.ds(q0 + 2 * QB, QB), :], buf0, sem0)

                h1.wait()
                res = _compute8(buf1, res, QB)
                out_v[pl.ds(pl.multiple_of(g * L, L), L)] = res * vfv
                return carry

            lax.fori_loop(0, NCHUNK // 2, _pair, 0)
            pltpu.sync_copy(out_v, out_hbm.at[pp])


PB_TC = 2


def _tc_body(valid_ref, costs_ref, occ_ref, out_ref):
    pid = pl.program_id(0)
    for i in range(PB_TC):
        c = costs_ref[pid * PB_TC + i]
        c = jnp.where(jnp.isnan(c), jnp.float32(1.0), c)
        vf = valid_ref[pid * PB_TC + i]
        out_ref[i] = jnp.sum(occ_ref[i] * c[None, :], axis=1)[None, :] * vf


def _tc_logits(costs, valid_f, occ):
    out = pl.pallas_call(
        _tc_body,
        grid=(P_TC // PB_TC,),
        in_specs=[
            pl.BlockSpec(memory_space=pltpu.SMEM),
            pl.BlockSpec((P, S), lambda p: (0, 0)),
            pl.BlockSpec((PB_TC, Q, S), lambda p: (p, 0, 0)),
        ],
        out_specs=pl.BlockSpec((PB_TC, 1, Q), lambda p: (p, 0, 0)),
        out_shape=jax.ShapeDtypeStruct((P_TC, 1, Q), jnp.float32),
    )(valid_f, costs, occ)
    return out.reshape(P_TC, Q)


def kernel(costs, valid, occ):
    valid_f = valid.astype(jnp.float32)
    sc_part = _sc_logits(costs, valid_f, occ)
    tc_part = _tc_logits(costs, valid_f, occ)
    logits = jnp.concatenate([tc_part, sc_part], axis=0)
    return (logits, valid)


# speedup vs baseline: 1.0175x; 1.0068x over previous
import functools

import jax
import jax.numpy as jnp
from jax import lax
from jax.experimental import pallas as pl
from jax.experimental.pallas import tpu as pltpu
from jax.experimental.pallas import tpu_sc as plsc

P, Q, S = 64, 64, 4096
L = 16
NC, NS = 2, 16
NW = NC * NS
P_TC = 34
P_SC = P - P_TC
QB = 8
NCHUNK = Q // QB
SCH = S // L

_mesh = plsc.VectorSubcoreMesh(core_axis_name="c", subcore_axis_name="s")


@functools.partial(
    pl.kernel,
    mesh=_mesh,
    out_type=jax.ShapeDtypeStruct((P_SC, Q), jnp.float32),
    compiler_params=pltpu.CompilerParams(needs_layout_passes=False),
    scratch_types=[
        pltpu.VMEM((S,), jnp.float32),
        pltpu.VMEM((P,), jnp.float32),
        pltpu.VMEM((Q,), jnp.float32),
        pltpu.VMEM((QB, S), jnp.float32),
        pltpu.VMEM((QB, S), jnp.float32),
        pltpu.SemaphoreType.DMA,
        pltpu.SemaphoreType.DMA,
    ],
)
def _sc_logits(costs_hbm, valid_hbm, occ_hbm, out_hbm,
               costs_v, valid_v, out_v, buf0, buf1, sem0, sem1):
    wid = lax.axis_index("s") * NC + lax.axis_index("c")
    pltpu.sync_copy(valid_hbm, valid_v)
    bufs = (buf0, buf1)
    sems = (sem0, sem1)
    lane_iota = lax.iota(jnp.int32, L)
    for t in range((P_SC + NW - 1) // NW):

        @pl.when(t * NW + wid < P_SC)
        def _problem():
            pp = t * NW + wid
            p = P_TC + pp
            pltpu.sync_copy(costs_hbm.at[p], costs_v)

            def _clean(si, carry):
                base = pl.multiple_of(si * L, L)
                c = costs_v[pl.ds(base, L)]
                costs_v[pl.ds(base, L)] = jnp.where(c != c, jnp.float32(1.0), c)
                return carry

            lax.fori_loop(0, SCH, _clean, 0)

            pbase = pl.multiple_of((p // L) * L, L)
            vvec = valid_v[pl.ds(pbase, L)]
            vfv = jnp.sum(jnp.where(lane_iota == (p - pbase), vvec,
                                    jnp.float32(0.0)))

            def _compute8(buf, res, off):
                def _acc(si, accs):
                    base = pl.multiple_of(si * L, L)
                    c = costs_v[pl.ds(base, L)]
                    return tuple(accs[j] + buf[j, pl.ds(base, L)] * c
                                 for j in range(QB))

                accs = lax.fori_loop(
                    0, SCH, _acc,
                    tuple(jnp.zeros((L,), jnp.float32) for _ in range(QB)))
                for j in range(QB):
                    res = jnp.where(lane_iota == (off + j),
                                    jnp.sum(accs[j]), res)
                return res

            pltpu.async_copy(occ_hbm.at[p, pl.ds(0, QB), :], buf0, sem0)

            def _pair(g, carry):
                q0 = pl.multiple_of(g * 2 * QB, 2 * QB)
                h1 = pltpu.async_copy(
                    occ_hbm.at[p, pl.ds(q0 + QB, QB), :], buf1, sem1)
                pltpu.make_async_copy(
                    occ_hbm.at[p, pl.ds(q0, QB), :], buf0, sem0).wait()
                res = _compute8(buf0, jnp.zeros((L,), jnp.float32), 0)

                @pl.when(g < NCHUNK // 2 - 1)
                def _prefetch():
                    pltpu.async_copy(
                        occ_hbm.at[p, pl.ds(q0 + 2 * QB, QB), :], buf0, sem0)

                h1.wait()
                res = _compute8(buf1, res, QB)
                out_v[pl.ds(pl.multiple_of(g * L, L), L)] = res * vfv
                return carry

            lax.fori_loop(0, NCHUNK // 2, _pair, 0)
            pltpu.sync_copy(out_v, out_hbm.at[pp])


PB_TC = 2


def _tc_body(valid_ref, costs_ref, occ_ref, out_ref):
    pid = pl.program_id(0)
    for i in range(PB_TC):
        c = costs_ref[pid * PB_TC + i]
        c = jnp.where(jnp.isnan(c), jnp.float32(1.0), c)
        vf = valid_ref[pid * PB_TC + i]
        out_ref[i] = jnp.sum(occ_ref[i] * c[None, :], axis=1)[None, :] * vf


def _tc_logits(costs, valid_f, occ):
    out = pl.pallas_call(
        _tc_body,
        grid=(P_TC // PB_TC,),
        in_specs=[
            pl.BlockSpec(memory_space=pltpu.SMEM),
            pl.BlockSpec((P, S), lambda p: (0, 0)),
            pl.BlockSpec((PB_TC, Q, S), lambda p: (p, 0, 0)),
        ],
        out_specs=pl.BlockSpec((PB_TC, 1, Q), lambda p: (p, 0, 0)),
        out_shape=jax.ShapeDtypeStruct((P_TC, 1, Q), jnp.float32),
    )(valid_f, costs, occ)
    return out.reshape(P_TC, Q)


def kernel(costs, valid, occ):
    valid_f = valid.astype(jnp.float32)
    sc_part = _sc_logits(costs, valid_f, occ)
    tc_part = _tc_logits(costs, valid_f, occ)
    logits = jnp.concatenate([tc_part, sc_part], axis=0)
    return (logits, valid)
